# baseline (speedup 1.0000x reference)
import functools

import jax
import jax.numpy as jnp
from jax import lax
from jax.experimental import pallas as pl
from jax.experimental.pallas import tpu as pltpu
from jax.experimental.pallas import tpu_sc as plsc

_NCYCLE = 12
_EMBED = 128
_BATCH = 16384
_NC = 2
_NS = 16
_NW = _NC * _NS
_BPW = _BATCH // _NW
_NK = _BPW // 128
_LANES = 16


@functools.partial(
    pl.kernel,
    mesh=plsc.VectorSubcoreMesh(core_axis_name="c", subcore_axis_name="s"),
    out_type=jax.ShapeDtypeStruct((_BATCH, _EMBED), jnp.float32),
    scratch_types=[
        pltpu.VMEM((_BPW,), jnp.float32),
        pltpu.VMEM((_NK, 128), jnp.int32),
        pltpu.VMEM((_BPW, _EMBED), jnp.float32),
        pltpu.SemaphoreType.DMA,
    ],
)
def _seasonal_embed(t_hbm, w_hbm, out_hbm, t_v, idx_v, rows_v, sem):
    wid = lax.axis_index("s") * _NC + lax.axis_index("c")
    base = wid * _BPW

    pltpu.sync_copy(t_hbm.at[pl.ds(base, _BPW)], t_v)

    for j in range(_NK):
        for c in range(128 // _LANES):
            x = t_v[pl.ds(j * 128 + c * _LANES, _LANES)] * jnp.float32(_NCYCLE)
            x = lax.rem(x, jnp.float32(_NCYCLE))
            idx_v[j, pl.ds(c * _LANES, _LANES)] = x.astype(jnp.int32)

    copies = [
        pltpu.async_copy(
            w_hbm.at[idx_v.at[j]], rows_v.at[pl.ds(j * 128, 128)], sem
        )
        for j in range(_NK)
    ]
    for cp in copies:
        cp.wait()

    pltpu.sync_copy(rows_v, out_hbm.at[pl.ds(base, _BPW)])


def kernel(t, W):
    return _seasonal_embed(t, W)


# speedup vs baseline: 3.2132x; 3.2132x over previous
import functools

import jax
import jax.numpy as jnp
from jax import lax
from jax.experimental import pallas as pl
from jax.experimental.pallas import tpu as pltpu
from jax.experimental.pallas import tpu_sc as plsc

_NCYCLE = 12
_EMBED = 128
_BATCH = 16384
_NC = 2
_NS = 16
_NW = _NC * _NS
_BPW = _BATCH // _NW
_NK = _BPW // 128
_LANES = 16


@functools.partial(
    pl.kernel,
    mesh=plsc.VectorSubcoreMesh(core_axis_name="c", subcore_axis_name="s"),
    out_type=jax.ShapeDtypeStruct((_BATCH, _EMBED), jnp.float32),
    scratch_types=[
        pltpu.VMEM((_BPW,), jnp.float32),
        pltpu.VMEM_SHARED((_NCYCLE, _EMBED), jnp.float32),
        pltpu.VMEM((_NK, 128), jnp.int32),
        pltpu.VMEM((_BPW, _EMBED), jnp.float32),
        pltpu.SemaphoreType.DMA,
    ],
)
def _seasonal_embed(t_hbm, w_hbm, out_hbm, t_v, w_sh, idx_v, rows_v, sem):
    sid = lax.axis_index("s")
    wid = sid * _NC + lax.axis_index("c")
    base = wid * _BPW

    @pl.when(sid == 0)
    def _():
        pltpu.sync_copy(w_hbm, w_sh)

    pltpu.sync_copy(t_hbm.at[pl.ds(base, _BPW)], t_v)
    plsc.subcore_barrier()

    for j in range(_NK):
        for c in range(128 // _LANES):
            x = t_v[pl.ds(j * 128 + c * _LANES, _LANES)] * jnp.float32(_NCYCLE)
            x = lax.rem(x, jnp.float32(_NCYCLE))
            idx_v[j, pl.ds(c * _LANES, _LANES)] = x.astype(jnp.int32)

    copies = [
        pltpu.async_copy(
            w_sh.at[idx_v.at[j]], rows_v.at[pl.ds(j * 128, 128)], sem
        )
        for j in range(_NK)
    ]
    for cp in copies:
        cp.wait()

    pltpu.sync_copy(rows_v, out_hbm.at[pl.ds(base, _BPW)])


def kernel(t, W):
    return _seasonal_embed(t, W)


# speedup vs baseline: 3.4424x; 1.0714x over previous
import functools

import jax
import jax.numpy as jnp
from jax import lax
from jax.experimental import pallas as pl
from jax.experimental.pallas import tpu as pltpu
from jax.experimental.pallas import tpu_sc as plsc

_NCYCLE = 12
_EMBED = 128
_BATCH = 16384
_NC = 2
_NS = 16
_NW = _NC * _NS
_BPW = _BATCH // _NW
_NK = _BPW // 128
_LANES = 16


@functools.partial(
    pl.kernel,
    mesh=plsc.VectorSubcoreMesh(core_axis_name="c", subcore_axis_name="s"),
    out_type=jax.ShapeDtypeStruct((_BATCH, _EMBED), jnp.float32),
    scratch_types=[
        pltpu.VMEM((_BPW,), jnp.float32),
        pltpu.VMEM_SHARED((_NCYCLE, _EMBED), jnp.float32),
        pltpu.VMEM((_NK, 128), jnp.int32),
        pltpu.VMEM((_BPW, _EMBED), jnp.float32),
        pltpu.SemaphoreType.DMA,
        pltpu.SemaphoreType.DMA,
        pltpu.SemaphoreType.DMA,
    ],
)
def _seasonal_embed(t_hbm, w_hbm, out_hbm, t_v, w_sh, idx_v, rows_v, tsem,
                    gsem, wsem):
    sid = lax.axis_index("s")
    wid = sid * _NC + lax.axis_index("c")
    base = wid * _BPW

    t_cp = pltpu.async_copy(t_hbm.at[pl.ds(base, _BPW)], t_v, tsem)

    @pl.when(sid == 0)
    def _():
        pltpu.sync_copy(w_hbm, w_sh)

    t_cp.wait()
    for j in range(_NK):
        for c in range(128 // _LANES):
            x = t_v[pl.ds(j * 128 + c * _LANES, _LANES)] * jnp.float32(_NCYCLE)
            x = lax.rem(x, jnp.float32(_NCYCLE))
            idx_v[j, pl.ds(c * _LANES, _LANES)] = x.astype(jnp.int32)

    plsc.subcore_barrier()

    gathers = [
        pltpu.async_copy(
            w_sh.at[idx_v.at[j]], rows_v.at[pl.ds(j * 128, 128)], gsem
        )
        for j in range(_NK)
    ]
    writes = []
    for j in range(_NK):
        gathers[j].wait()
        writes.append(
            pltpu.async_copy(
                rows_v.at[pl.ds(j * 128, 128)],
                out_hbm.at[pl.ds(base + j * 128, 128)],
                wsem,
            )
        )
    for cp in writes:
        cp.wait()


def kernel(t, W):
    return _seasonal_embed(t, W)
